# baseline (speedup 1.0000x reference)
import jax
import jax.numpy as jnp
from jax import lax
from jax.experimental import pallas as pl
from jax.experimental.pallas import tpu as pltpu
from jax.experimental.pallas import tpu_sc as plsc

VOCAB = 1000000
N_ZONAS = 52
B = 4096
L = 200
TOKENS = B * L
NUM_WORKERS = 32
PER_WORKER = TOKENS // NUM_WORKERS
CHUNK = 1024
CHUNKS = PER_WORKER // CHUNK
GATHER_SPLIT = CHUNK // 128


def _sc_kernel(table_hbm, ids_hbm, out_hbm, ids_v, idx_v, rows_u8, rows_v,
               out_v, sem):
    wid = lax.axis_index("s") * 2 + lax.axis_index("c")
    iota16 = lax.iota(jnp.int32, 16)

    def chunk_body(c, carry):
        base = (wid * CHUNKS + c) * CHUNK
        pltpu.sync_copy(ids_hbm.at[pl.ds(base, CHUNK)], ids_v)
        for j in range(GATHER_SPLIT):
            for k in range(8):
                s = ids_v[pl.ds(j * 128 + k * 16, 16)] >> 2
                idx_v[j, pl.ds(k * 16, 16)] = s
        descs = []
        for j in range(GATHER_SPLIT):
            d = pltpu.async_copy(
                table_hbm.at[idx_v.at[j]],
                rows_u8.at[pl.ds(j * 128, 128)],
                sem,
            )
            descs.append(d)
        for d in descs:
            d.wait()

        def pack_body(i, carry2):
            for k in range(4):
                r = i * 4 + k
                lo = plsc.bitcast(rows_u8[r, pl.ds(0, 64)], jnp.int32)
                hi = plsc.bitcast(rows_u8[r, pl.ds(64, 64)], jnp.int32)
                rows_v[r, pl.ds(0, 16)] = lo
                rows_v[r, pl.ds(16, 16)] = hi
            return carry2

        lax.fori_loop(0, CHUNK // 4, pack_body, 0)

        def group_body(g, carry2):
            rid = g * 16 + iota16
            ids_vec = ids_v[pl.ds(g * 16, 16)]
            word0 = (ids_vec & 3) << 3
            out_base = rid * N_ZONAS
            for g2 in range(N_ZONAS):
                w = word0 + (g2 >> 3)
                sh = (g2 & 7) << 2
                val = plsc.load_gather(rows_v, [rid, w])
                nib = (val >> sh) & 15
                f = nib.astype(jnp.float32) * (1.0 / 15.0)
                plsc.store_scatter(out_v, [out_base + g2], f)
            return carry2

        lax.fori_loop(0, CHUNK // 16, group_body, 0)
        pltpu.sync_copy(out_v, out_hbm.at[pl.ds(base * N_ZONAS, CHUNK * N_ZONAS)])
        return carry

    lax.fori_loop(0, CHUNKS, chunk_body, 0)


@jax.jit
def kernel(token_ids, tabla_cuant):
    flat_ids = token_ids.reshape(-1)
    packed = jnp.pad(tabla_cuant, ((0, 0), (0, 6))).reshape(VOCAB // 4, 128)
    mesh = plsc.VectorSubcoreMesh(core_axis_name="c", subcore_axis_name="s")
    out = pl.kernel(
        _sc_kernel,
        out_type=jax.ShapeDtypeStruct((TOKENS * N_ZONAS,), jnp.float32),
        mesh=mesh,
        scratch_types=[
            pltpu.VMEM((CHUNK,), jnp.int32),
            pltpu.VMEM((GATHER_SPLIT, 128), jnp.int32),
            pltpu.VMEM((CHUNK, 128), jnp.uint8),
            pltpu.VMEM((CHUNK, 32), jnp.int32),
            pltpu.VMEM((CHUNK * N_ZONAS,), jnp.float32),
            pltpu.SemaphoreType.DMA,
        ],
        compiler_params=pltpu.CompilerParams(
            needs_layout_passes=False, use_tc_tiling_on_sc=False
        ),
    )(packed, flat_ids)
    return out.reshape(B, L, N_ZONAS)


# speedup vs baseline: 1.0126x; 1.0126x over previous
import jax
import jax.numpy as jnp
from jax import lax
from jax.experimental import pallas as pl
from jax.experimental.pallas import tpu as pltpu
from jax.experimental.pallas import tpu_sc as plsc

VOCAB = 1000000
N_ZONAS = 52
B = 4096
L = 200
TOKENS = B * L
NUM_WORKERS = 32
PER_WORKER = TOKENS // NUM_WORKERS
CHUNK = 512
CHUNKS = PER_WORKER // CHUNK
GATHER_SPLIT = CHUNK // 128


def _sc_kernel(table_hbm, ids_hbm, out_hbm, ids_v, idx_v, rows_a, rows_b,
               cols_v, out_v, sem_a, sem_b):
    wid = lax.axis_index("s") * 2 + lax.axis_index("c")
    iota16 = lax.iota(jnp.int32, 16)
    worker_base = wid * CHUNKS

    def fire(c, buf, rows, sem):
        base = (worker_base + jnp.minimum(c, CHUNKS - 1)) * CHUNK
        pltpu.sync_copy(ids_hbm.at[pl.ds(base, CHUNK)], ids_v.at[buf])
        for j in range(GATHER_SPLIT):
            for k in range(8):
                s = ids_v[buf, pl.ds(j * 128 + k * 16, 16)] >> 2
                idx_v[buf, j, pl.ds(k * 16, 16)] = s
        for j in range(GATHER_SPLIT):
            pltpu.async_copy(
                table_hbm.at[idx_v.at[buf, j]],
                rows.at[pl.ds(j * 128, 128)],
                sem,
            )

    def drain(buf, rows, sem):
        for j in range(GATHER_SPLIT):
            pltpu.make_async_copy(
                table_hbm.at[idx_v.at[buf, j]],
                rows.at[pl.ds(j * 128, 128)],
                sem,
            ).wait()

    def process(c, buf, rows):
        def pack_body(i, carry):
            for k in range(4):
                r = i * 4 + k
                lo = plsc.bitcast(rows[r, pl.ds(0, 64)], jnp.int32)
                hi = plsc.bitcast(rows[r, pl.ds(64, 64)], jnp.int32)
                cols_v[r, pl.ds(0, 16)] = lo
                cols_v[r, pl.ds(16, 16)] = hi
            return carry

        lax.fori_loop(0, CHUNK // 4, pack_body, 0)

        def group_body(g, carry):
            rid = g * 16 + iota16
            ids_vec = ids_v[buf, pl.ds(g * 16, 16)]
            word0 = (ids_vec & 3) << 3
            out_base = rid * N_ZONAS
            for g2 in range(N_ZONAS):
                w = word0 + (g2 >> 3)
                sh = (g2 & 7) << 2
                val = plsc.load_gather(cols_v, [rid, w])
                nib = (val >> sh) & 15
                f = nib.astype(jnp.float32) * (1.0 / 15.0)
                plsc.store_scatter(out_v, [out_base + g2], f)
            return carry

        lax.fori_loop(0, CHUNK // 16, group_body, 0)
        base = (worker_base + c) * CHUNK
        pltpu.sync_copy(out_v, out_hbm.at[pl.ds(base * N_ZONAS, CHUNK * N_ZONAS)])

    fire(0, 0, rows_a, sem_a)

    def pair_body(p, carry):
        c = p * 2
        fire(c + 1, 1, rows_b, sem_b)
        drain(0, rows_a, sem_a)
        process(c, 0, rows_a)
        fire(c + 2, 0, rows_a, sem_a)
        drain(1, rows_b, sem_b)
        process(c + 1, 1, rows_b)
        return carry

    lax.fori_loop(0, CHUNKS // 2, pair_body, 0)
    drain(0, rows_a, sem_a)


@jax.jit
def kernel(token_ids, tabla_cuant):
    flat_ids = token_ids.reshape(-1)
    packed = jnp.pad(tabla_cuant, ((0, 0), (0, 6))).reshape(VOCAB // 4, 128)
    mesh = plsc.VectorSubcoreMesh(core_axis_name="c", subcore_axis_name="s")
    out = pl.kernel(
        _sc_kernel,
        out_type=jax.ShapeDtypeStruct((TOKENS * N_ZONAS,), jnp.float32),
        mesh=mesh,
        scratch_types=[
            pltpu.VMEM((2, CHUNK), jnp.int32),
            pltpu.VMEM((2, GATHER_SPLIT, 128), jnp.int32),
            pltpu.VMEM((CHUNK, 128), jnp.uint8),
            pltpu.VMEM((CHUNK, 128), jnp.uint8),
            pltpu.VMEM((CHUNK, 32), jnp.int32),
            pltpu.VMEM((CHUNK * N_ZONAS,), jnp.float32),
            pltpu.SemaphoreType.DMA,
            pltpu.SemaphoreType.DMA,
        ],
        compiler_params=pltpu.CompilerParams(
            needs_layout_passes=False, use_tc_tiling_on_sc=False
        ),
    )(packed, flat_ids)
    return out.reshape(B, L, N_ZONAS)


# speedup vs baseline: 1.4417x; 1.4238x over previous
import jax
import jax.numpy as jnp
from jax import lax
from jax.experimental import pallas as pl
from jax.experimental.pallas import tpu as pltpu
from jax.experimental.pallas import tpu_sc as plsc

VOCAB = 1000000
N_ZONAS = 52
B = 4096
L = 200
TOKENS = B * L
NUM_WORKERS = 32
PER_WORKER = TOKENS // NUM_WORKERS
CHUNK = 512
CHUNKS = PER_WORKER // CHUNK
GATHER_SPLIT = CHUNK // 128


def _sc_kernel(table_hbm, ids_hbm, out_hbm, ids_v, idx_v, rows_a, rows_b,
               cols_v, out_v, sem_a, sem_b, osem):
    wid = lax.axis_index("s") * 2 + lax.axis_index("c")
    iota16 = lax.iota(jnp.int32, 16)
    worker_base = wid * CHUNKS

    def fire(c, buf, rows, sem):
        base = (worker_base + jnp.minimum(c, CHUNKS - 1)) * CHUNK
        pltpu.sync_copy(ids_hbm.at[pl.ds(base, CHUNK)], ids_v.at[buf])
        for j in range(GATHER_SPLIT):
            for k in range(8):
                s = ids_v[buf, pl.ds(j * 128 + k * 16, 16)] >> 2
                idx_v[buf, j, pl.ds(k * 16, 16)] = s
        for j in range(GATHER_SPLIT):
            pltpu.async_copy(
                table_hbm.at[idx_v.at[buf, j]],
                rows.at[pl.ds(j * 128, 128)],
                sem,
            )

    def drain(buf, rows, sem):
        for j in range(GATHER_SPLIT):
            pltpu.make_async_copy(
                table_hbm.at[idx_v.at[buf, j]],
                rows.at[pl.ds(j * 128, 128)],
                sem,
            ).wait()

    def process(c, buf, rows):
        def pack_body(i, carry):
            for k in range(4):
                r = i * 4 + k
                lo = plsc.bitcast(rows[r, pl.ds(0, 64)], jnp.int32)
                hi = plsc.bitcast(rows[r, pl.ds(64, 64)], jnp.int32)
                cols_v[r, pl.ds(0, 16)] = lo
                cols_v[r, pl.ds(16, 16)] = hi
            return carry

        lax.fori_loop(0, CHUNK // 4, pack_body, 0)

        def group_body(g, carry):
            rid = g * 16 + iota16
            ids_vec = ids_v[buf, pl.ds(g * 16, 16)]
            word0 = (ids_vec & 3) << 3
            out_base = rid * N_ZONAS
            for w in range(7):
                val = plsc.load_gather(cols_v, [rid, word0 + w])
                for n in range(8 if w < 6 else 4):
                    nib = (val >> (4 * n)) & 15
                    f = nib.astype(jnp.float32) * (1.0 / 15.0)
                    plsc.store_scatter(out_v.at[buf], [out_base + (8 * w + n)], f)
            return carry

        lax.fori_loop(0, CHUNK // 16, group_body, 0)
        base = (worker_base + c) * CHUNK
        pltpu.async_copy(
            out_v.at[buf],
            out_hbm.at[pl.ds(base * N_ZONAS, CHUNK * N_ZONAS)],
            osem,
        )

    def out_drain(c, buf):
        base = (worker_base + c) * CHUNK
        pltpu.make_async_copy(
            out_v.at[buf],
            out_hbm.at[pl.ds(base * N_ZONAS, CHUNK * N_ZONAS)],
            osem,
        ).wait()

    fire(0, 0, rows_a, sem_a)

    def pair_body(p, carry):
        c = p * 2
        fire(c + 1, 1, rows_b, sem_b)
        drain(0, rows_a, sem_a)

        @pl.when(p > 0)
        def _():
            out_drain(c - 2, 0)

        process(c, 0, rows_a)
        fire(c + 2, 0, rows_a, sem_a)
        drain(1, rows_b, sem_b)

        @pl.when(p > 0)
        def _():
            out_drain(c - 1, 1)

        process(c + 1, 1, rows_b)
        return carry

    lax.fori_loop(0, CHUNKS // 2, pair_body, 0)
    drain(0, rows_a, sem_a)
    out_drain(CHUNKS - 2, 0)
    out_drain(CHUNKS - 1, 1)


@jax.jit
def kernel(token_ids, tabla_cuant):
    flat_ids = token_ids.reshape(-1)
    packed = jnp.pad(tabla_cuant, ((0, 0), (0, 6))).reshape(VOCAB // 4, 128)
    mesh = plsc.VectorSubcoreMesh(core_axis_name="c", subcore_axis_name="s")
    out = pl.kernel(
        _sc_kernel,
        out_type=jax.ShapeDtypeStruct((TOKENS * N_ZONAS,), jnp.float32),
        mesh=mesh,
        scratch_types=[
            pltpu.VMEM((2, CHUNK), jnp.int32),
            pltpu.VMEM((2, GATHER_SPLIT, 128), jnp.int32),
            pltpu.VMEM((CHUNK, 128), jnp.uint8),
            pltpu.VMEM((CHUNK, 128), jnp.uint8),
            pltpu.VMEM((CHUNK, 32), jnp.int32),
            pltpu.VMEM((2, CHUNK * N_ZONAS), jnp.float32),
            pltpu.SemaphoreType.DMA,
            pltpu.SemaphoreType.DMA,
            pltpu.SemaphoreType.DMA,
        ],
        compiler_params=pltpu.CompilerParams(
            needs_layout_passes=False, use_tc_tiling_on_sc=False
        ),
    )(packed, flat_ids)
    return out.reshape(B, L, N_ZONAS)
